# Initial kernel scaffold; baseline (speedup 1.0000x reference)
import jax
import jax.numpy as jnp
from jax.experimental import pallas as pl


def kernel(measurements, measurement_to_tag_mapping, w1, b1, w2, b2, wq, bq, wk, bk, wv, bv, wa1a, ba1a, wa1b, ba1b, wa2a, ba2a, wa2b, ba2b, wmu, bmu, wl, bl):
    raise NotImplementedError("write your pallas kernel here")


# speedup vs baseline: 1.4569x; 1.4569x over previous
import jax
import jax.numpy as jnp
from jax import lax
from jax.experimental import pallas as pl
from jax.experimental.pallas import tpu as pltpu

H = 64
T = 64
NEG = -1e9
QCHUNK = 512


def _body(x_ref, map_ref, w1_ref, b1_ref, w2_ref, b2_ref, wq_ref, bq_ref,
          wk_ref, bk_ref, wv_ref, bv_ref, wa1a_ref, ba1a_ref, wa1b_ref,
          ba1b_ref, wa2a_ref, ba2a_ref, wa2b_ref, ba2b_ref, wo_ref, bo_ref,
          out_ref):
    x = x_ref[0]
    n = x.shape[0]

    f = jnp.maximum(jnp.dot(x, w1_ref[...]) + b1_ref[...], 0.0)
    f = jnp.maximum(jnp.dot(f, w2_ref[...]) + b2_ref[...], 0.0)

    q = jnp.dot(f, wq_ref[...]) + bq_ref[...]
    k = jnp.dot(f, wk_ref[...]) + bk_ref[...]
    v = jnp.dot(f, wv_ref[...]) + bv_ref[...]

    outs = []
    for i in range(n // QCHUNK):
        qc = q[i * QCHUNK:(i + 1) * QCHUNK]
        s = lax.dot_general(qc, k, (((1,), (1,)), ((), ()))) * 0.125
        s = s - jnp.max(s, axis=1, keepdims=True)
        p = jnp.exp(s)
        num = lax.dot_general(p, v, (((1,), (0,)), ((), ())))
        outs.append(num / jnp.sum(p, axis=1, keepdims=True))
    f = f + jnp.concatenate(outs, axis=0)

    hid = jnp.maximum(jnp.dot(f, wa1a_ref[...]) + ba1a_ref[...], 0.0)
    s1 = jnp.dot(hid, wa1b_ref[...]) + ba1b_ref[...]

    tags = map_ref[0]
    col = lax.broadcasted_iota(jnp.int32, (n, T), 1)
    mask = tags == col
    maskf = mask.astype(jnp.float32)
    masked = jnp.where(mask, s1, NEG)
    m0 = jnp.max(masked, axis=0, keepdims=True)
    p0 = jnp.exp(masked - m0)
    denom = jnp.sum(p0, axis=0, keepdims=True)
    w_seg = p0 * maskf / denom

    tagf = lax.dot_general(w_seg, f, (((0,), (0,)), ((), ())))
    ones = jnp.ones((n, 1), jnp.float32)
    cnt = lax.dot_general(maskf, ones, (((0,), (0,)), ((), ())))

    hid2 = jnp.maximum(jnp.dot(tagf, wa2a_ref[...]) + ba2a_ref[...], 0.0)
    s2 = jnp.dot(hid2, wa2b_ref[...]) + ba2b_ref[...]
    masked2 = jnp.where(cnt > 0.5, s2, NEG)
    m2 = jnp.max(masked2, axis=0, keepdims=True)
    p2 = jnp.exp(masked2 - m2)
    w2m = p2 / jnp.sum(p2, axis=0, keepdims=True)

    pose = lax.dot_general(w2m, tagf, (((0,), (0,)), ((), ())))
    out_ref[0] = jnp.dot(pose, wo_ref[...]) + bo_ref[...]


def kernel(measurements, measurement_to_tag_mapping, w1, b1, w2, b2, wq, bq,
           wk, bk, wv, bv, wa1a, ba1a, wa1b, ba1b, wa2a, ba2a, wa2b, ba2b,
           wmu, bmu, wl, bl):
    B, N, F = measurements.shape
    xp = jnp.concatenate(
        [measurements, jnp.zeros((B, N, 8 - F), jnp.float32)], axis=-1)
    w1p = jnp.concatenate([w1, jnp.zeros((8 - F, H), jnp.float32)], axis=0)
    mapc = measurement_to_tag_mapping.reshape(B, N, 1)

    wo = jnp.concatenate([wmu, wl], axis=1)
    wo = jnp.concatenate([wo, jnp.zeros((H, 128 - 9), jnp.float32)], axis=1)
    bo = jnp.concatenate([bmu, bl, jnp.zeros((128 - 9,), jnp.float32)])
    bo = bo.reshape(1, 128)

    row = lambda a: a.reshape(1, -1)
    full = lambda shp: pl.BlockSpec(shp, lambda b: (0,) * len(shp))

    out = pl.pallas_call(
        _body,
        grid=(B,),
        in_specs=[
            pl.BlockSpec((1, N, 8), lambda b: (b, 0, 0)),
            pl.BlockSpec((1, N, 1), lambda b: (b, 0, 0)),
            full((8, H)), full((1, H)),
            full((H, H)), full((1, H)),
            full((H, H)), full((1, H)),
            full((H, H)), full((1, H)),
            full((H, H)), full((1, H)),
            full((H, H // 2)), full((1, H // 2)),
            full((H // 2, 1)), full((1, 1)),
            full((H, H // 2)), full((1, H // 2)),
            full((H // 2, 1)), full((1, 1)),
            full((H, 128)), full((1, 128)),
        ],
        out_specs=pl.BlockSpec((1, 1, 128), lambda b: (b, 0, 0)),
        out_shape=jax.ShapeDtypeStruct((B, 1, 128), jnp.float32),
        compiler_params=pltpu.CompilerParams(
            dimension_semantics=("arbitrary",)),
    )(xp, mapc, w1p, row(b1), w2, row(b2), wq, row(bq), wk, row(bk),
      wv, row(bv), wa1a, row(ba1a), wa1b, row(ba1b), wa2a, row(ba2a),
      wa2b, row(ba2b), wo, bo)

    return (out[:, 0, :3], out[:, 0, 3:9])


# speedup vs baseline: 1.5896x; 1.0911x over previous
import jax
import jax.numpy as jnp
from jax import lax
from jax.experimental import pallas as pl
from jax.experimental.pallas import tpu as pltpu

H = 64
T = 64
NEG = -1e9
QCHUNK = 512


def _body(x_ref, map_ref, w1_ref, b1_ref, w2_ref, b2_ref, wq_ref, bq_ref,
          wk_ref, bk_ref, wv_ref, bv_ref, wa1a_ref, ba1a_ref, wa1b_ref,
          ba1b_ref, wa2a_ref, ba2a_ref, wa2b_ref, ba2b_ref, wo_ref, bo_ref,
          out_ref):
    x = x_ref[0]
    n = x.shape[0]

    f = jnp.maximum(jnp.dot(x, w1_ref[...]) + b1_ref[...], 0.0)
    f = jnp.maximum(jnp.dot(f, w2_ref[...]) + b2_ref[...], 0.0)

    LOG2E = 1.4426950408889634
    q = (jnp.dot(f, wq_ref[...]) + bq_ref[...]) * (0.125 * LOG2E)
    k = jnp.dot(f, wk_ref[...]) + bk_ref[...]
    v = jnp.dot(f, wv_ref[...]) + bv_ref[...]
    v_aug = jnp.concatenate([v, jnp.ones((n, 1), jnp.float32)], axis=1)

    outs = []
    for i in range(n // QCHUNK):
        qc = q[i * QCHUNK:(i + 1) * QCHUNK]
        s = lax.dot_general(qc, k, (((1,), (1,)), ((), ())))
        p = jnp.exp2(s - jnp.max(s, axis=1, keepdims=True))
        acc = lax.dot_general(p, v_aug, (((1,), (0,)), ((), ())))
        outs.append(acc[:, :H] / acc[:, H:H + 1])
    f = f + jnp.concatenate(outs, axis=0)

    hid = jnp.maximum(jnp.dot(f, wa1a_ref[...]) + ba1a_ref[...], 0.0)
    s1 = jnp.dot(hid, wa1b_ref[...]) + ba1b_ref[...]

    tags = map_ref[0]
    col = lax.broadcasted_iota(jnp.int32, (n, T), 1)
    mask = tags == col
    maskf = mask.astype(jnp.float32)
    masked = jnp.where(mask, s1, NEG)
    m0 = jnp.max(masked, axis=0, keepdims=True)
    p0 = jnp.exp(masked - m0)
    denom = jnp.sum(p0, axis=0, keepdims=True)
    w_seg = p0 * maskf / denom

    tagf = lax.dot_general(w_seg, f, (((0,), (0,)), ((), ())))
    ones = jnp.ones((n, 1), jnp.float32)
    cnt = lax.dot_general(maskf, ones, (((0,), (0,)), ((), ())))

    hid2 = jnp.maximum(jnp.dot(tagf, wa2a_ref[...]) + ba2a_ref[...], 0.0)
    s2 = jnp.dot(hid2, wa2b_ref[...]) + ba2b_ref[...]
    masked2 = jnp.where(cnt > 0.5, s2, NEG)
    m2 = jnp.max(masked2, axis=0, keepdims=True)
    p2 = jnp.exp(masked2 - m2)
    w2m = p2 / jnp.sum(p2, axis=0, keepdims=True)

    pose = lax.dot_general(w2m, tagf, (((0,), (0,)), ((), ())))
    out_ref[0] = jnp.dot(pose, wo_ref[...]) + bo_ref[...]


def kernel(measurements, measurement_to_tag_mapping, w1, b1, w2, b2, wq, bq,
           wk, bk, wv, bv, wa1a, ba1a, wa1b, ba1b, wa2a, ba2a, wa2b, ba2b,
           wmu, bmu, wl, bl):
    B, N, F = measurements.shape
    xp = jnp.concatenate(
        [measurements, jnp.zeros((B, N, 8 - F), jnp.float32)], axis=-1)
    w1p = jnp.concatenate([w1, jnp.zeros((8 - F, H), jnp.float32)], axis=0)
    mapc = measurement_to_tag_mapping.reshape(B, N, 1)

    wo = jnp.concatenate([wmu, wl], axis=1)
    wo = jnp.concatenate([wo, jnp.zeros((H, 128 - 9), jnp.float32)], axis=1)
    bo = jnp.concatenate([bmu, bl, jnp.zeros((128 - 9,), jnp.float32)])
    bo = bo.reshape(1, 128)

    row = lambda a: a.reshape(1, -1)
    full = lambda shp: pl.BlockSpec(shp, lambda b: (0,) * len(shp))

    out = pl.pallas_call(
        _body,
        grid=(B,),
        in_specs=[
            pl.BlockSpec((1, N, 8), lambda b: (b, 0, 0)),
            pl.BlockSpec((1, N, 1), lambda b: (b, 0, 0)),
            full((8, H)), full((1, H)),
            full((H, H)), full((1, H)),
            full((H, H)), full((1, H)),
            full((H, H)), full((1, H)),
            full((H, H)), full((1, H)),
            full((H, H // 2)), full((1, H // 2)),
            full((H // 2, 1)), full((1, 1)),
            full((H, H // 2)), full((1, H // 2)),
            full((H // 2, 1)), full((1, 1)),
            full((H, 128)), full((1, 128)),
        ],
        out_specs=pl.BlockSpec((1, 1, 128), lambda b: (b, 0, 0)),
        out_shape=jax.ShapeDtypeStruct((B, 1, 128), jnp.float32),
        compiler_params=pltpu.CompilerParams(
            dimension_semantics=("parallel",)),
    )(xp, mapc, w1p, row(b1), w2, row(b2), wq, row(bq), wk, row(bk),
      wv, row(bv), wa1a, row(ba1a), wa1b, row(ba1b), wa2a, row(ba2a),
      wa2b, row(ba2b), wo, bo)

    return (out[:, 0, :3], out[:, 0, 3:9])


# speedup vs baseline: 1.6075x; 1.0113x over previous
import jax
import jax.numpy as jnp
from jax import lax
from jax.experimental import pallas as pl
from jax.experimental.pallas import tpu as pltpu

H = 64
T = 64
NEG = -1e9
QCHUNK = 512


def _body(x_ref, map_ref, w1_ref, b1_ref, w2_ref, b2_ref, wq_ref, bq_ref,
          wk_ref, bk_ref, wv_ref, bv_ref, wa1a_ref, ba1a_ref, wa1b_ref,
          ba1b_ref, wa2a_ref, ba2a_ref, wa2b_ref, ba2b_ref, wo_ref, bo_ref,
          out_ref):
    x = x_ref[0]
    n = x.shape[0]

    f = jnp.maximum(jnp.dot(x, w1_ref[...]) + b1_ref[...], 0.0)
    f = jnp.maximum(jnp.dot(f, w2_ref[...]) + b2_ref[...], 0.0)

    LOG2E = 1.4426950408889634
    q = (jnp.dot(f, wq_ref[...]) + bq_ref[...]) * (0.125 * LOG2E)
    k = jnp.dot(f, wk_ref[...]) + bk_ref[...]
    v = jnp.dot(f, wv_ref[...]) + bv_ref[...]
    qsq = jnp.sum(q * q, axis=1, keepdims=True)
    ksq = jnp.max(jnp.sum(k * k, axis=1, keepdims=True))
    bound = jnp.sqrt(qsq * ksq)
    v_aug = jnp.concatenate(
        [v, jnp.ones((n, 1), jnp.float32)], axis=1).astype(jnp.bfloat16)
    qb = q.astype(jnp.bfloat16)
    kb = k.astype(jnp.bfloat16)

    outs = []
    for i in range(n // QCHUNK):
        qc = qb[i * QCHUNK:(i + 1) * QCHUNK]
        s = lax.dot_general(qc, kb, (((1,), (1,)), ((), ())),
                            preferred_element_type=jnp.float32)
        p = jnp.exp2(
            s - bound[i * QCHUNK:(i + 1) * QCHUNK]).astype(jnp.bfloat16)
        acc = lax.dot_general(p, v_aug, (((1,), (0,)), ((), ())),
                              preferred_element_type=jnp.float32)
        outs.append(acc[:, :H] / acc[:, H:H + 1])
    f = f + jnp.concatenate(outs, axis=0)

    hid = jnp.maximum(jnp.dot(f, wa1a_ref[...]) + ba1a_ref[...], 0.0)
    s1 = jnp.dot(hid, wa1b_ref[...]) + ba1b_ref[...]

    tags = map_ref[0]
    col = lax.broadcasted_iota(jnp.int32, (n, T), 1)
    mask = tags == col
    maskf = mask.astype(jnp.float32)
    masked = jnp.where(mask, s1, NEG)
    m0 = jnp.max(masked, axis=0, keepdims=True)
    p0 = jnp.exp(masked - m0)
    denom = jnp.sum(p0, axis=0, keepdims=True)
    w_seg = p0 * maskf / denom

    tagf = lax.dot_general(w_seg, f, (((0,), (0,)), ((), ())))
    ones = jnp.ones((n, 1), jnp.float32)
    cnt = lax.dot_general(maskf, ones, (((0,), (0,)), ((), ())))

    hid2 = jnp.maximum(jnp.dot(tagf, wa2a_ref[...]) + ba2a_ref[...], 0.0)
    s2 = jnp.dot(hid2, wa2b_ref[...]) + ba2b_ref[...]
    masked2 = jnp.where(cnt > 0.5, s2, NEG)
    m2 = jnp.max(masked2, axis=0, keepdims=True)
    p2 = jnp.exp(masked2 - m2)
    w2m = p2 / jnp.sum(p2, axis=0, keepdims=True)

    pose = lax.dot_general(w2m, tagf, (((0,), (0,)), ((), ())))
    out_ref[0] = jnp.dot(pose, wo_ref[...]) + bo_ref[...]


def kernel(measurements, measurement_to_tag_mapping, w1, b1, w2, b2, wq, bq,
           wk, bk, wv, bv, wa1a, ba1a, wa1b, ba1b, wa2a, ba2a, wa2b, ba2b,
           wmu, bmu, wl, bl):
    B, N, F = measurements.shape
    xp = jnp.concatenate(
        [measurements, jnp.zeros((B, N, 8 - F), jnp.float32)], axis=-1)
    w1p = jnp.concatenate([w1, jnp.zeros((8 - F, H), jnp.float32)], axis=0)
    mapc = measurement_to_tag_mapping.reshape(B, N, 1)

    wo = jnp.concatenate([wmu, wl], axis=1)
    wo = jnp.concatenate([wo, jnp.zeros((H, 128 - 9), jnp.float32)], axis=1)
    bo = jnp.concatenate([bmu, bl, jnp.zeros((128 - 9,), jnp.float32)])
    bo = bo.reshape(1, 128)

    row = lambda a: a.reshape(1, -1)
    full = lambda shp: pl.BlockSpec(shp, lambda b: (0,) * len(shp))

    out = pl.pallas_call(
        _body,
        grid=(B,),
        in_specs=[
            pl.BlockSpec((1, N, 8), lambda b: (b, 0, 0)),
            pl.BlockSpec((1, N, 1), lambda b: (b, 0, 0)),
            full((8, H)), full((1, H)),
            full((H, H)), full((1, H)),
            full((H, H)), full((1, H)),
            full((H, H)), full((1, H)),
            full((H, H)), full((1, H)),
            full((H, H // 2)), full((1, H // 2)),
            full((H // 2, 1)), full((1, 1)),
            full((H, H // 2)), full((1, H // 2)),
            full((H // 2, 1)), full((1, 1)),
            full((H, 128)), full((1, 128)),
        ],
        out_specs=pl.BlockSpec((1, 1, 128), lambda b: (b, 0, 0)),
        out_shape=jax.ShapeDtypeStruct((B, 1, 128), jnp.float32),
        compiler_params=pltpu.CompilerParams(
            dimension_semantics=("parallel",)),
    )(xp, mapc, w1p, row(b1), w2, row(b2), wq, row(bq), wk, row(bk),
      wv, row(bv), wa1a, row(ba1a), wa1b, row(ba1b), wa2a, row(ba2a),
      wa2b, row(ba2b), wo, bo)

    return (out[:, 0, :3], out[:, 0, 3:9])
